# baseline (speedup 1.0000x reference)
import jax
import jax.numpy as jnp
from jax.experimental import pallas as pl
from jax.experimental.pallas import tpu as pltpu
from jax.experimental import compute_on

_BM = 400
_SC_ROWS = 1200


def _mm_kernel(mask_ref, oh_ref, out_ref):
    out_ref[...] = jnp.dot(mask_ref[...], oh_ref[...],
                           preferred_element_type=jnp.float32)


def kernel(mask_matrix, x, one_hot_h):
    del x
    n_rows, k = mask_matrix.shape
    n_types = one_hot_h.shape[1]
    m = n_rows - _SC_ROWS

    @compute_on.compute_on('tpu_sparsecore')
    @jax.jit
    def _sc_head(mm, oh):
        return jnp.matmul(mm, oh)

    head = _sc_head(mask_matrix[:_SC_ROWS], one_hot_h)
    tail = pl.pallas_call(
        _mm_kernel,
        grid=(pl.cdiv(m, _BM),),
        in_specs=[
            pl.BlockSpec((_BM, k), lambda i: (i + _SC_ROWS // _BM, 0)),
            pl.BlockSpec((k, n_types), lambda i: (0, 0)),
        ],
        out_specs=pl.BlockSpec((_BM, n_types), lambda i: (i, 0)),
        out_shape=jax.ShapeDtypeStruct((m, n_types), jnp.float32),
        compiler_params=pltpu.CompilerParams(
            dimension_semantics=("arbitrary",),
        ),
    )(mask_matrix, one_hot_h)
    return jnp.concatenate([head, tail], axis=0)


# speedup vs baseline: 1.0104x; 1.0104x over previous
import jax
import jax.numpy as jnp
from jax.experimental import pallas as pl
from jax.experimental.pallas import tpu as pltpu

_BM = 400


def _mm_kernel(mask_ref, oh_ref, out_ref):
    out_ref[...] = jnp.dot(mask_ref[...], oh_ref[...],
                           preferred_element_type=jnp.float32)


def kernel(mask_matrix, x, one_hot_h):
    del x
    n_rows, k = mask_matrix.shape
    n_types = one_hot_h.shape[1]
    return pl.pallas_call(
        _mm_kernel,
        grid=(n_rows // _BM,),
        in_specs=[
            pl.BlockSpec((_BM, k), lambda i: (i, 0)),
            pl.BlockSpec((k, n_types), lambda i: (0, 0)),
        ],
        out_specs=pl.BlockSpec((_BM, n_types), lambda i: (i, 0)),
        out_shape=jax.ShapeDtypeStruct((n_rows, n_types), jnp.float32),
        compiler_params=pltpu.CompilerParams(
            dimension_semantics=("arbitrary",),
        ),
    )(mask_matrix, one_hot_h)
